# baseline (speedup 1.0000x reference)
import functools

import jax
import jax.numpy as jnp
from jax import lax
from jax.experimental import pallas as pl
from jax.experimental.pallas import tpu as pltpu
from jax.experimental.pallas import tpu_sc as plsc

_D = 256
_B = 16
_N = 1024
_HW = 32
_CHUNK = 128
_LANES = 16
_NC = 2
_NS = 16
_NW = _NC * _NS
_TOK_PER_W = (_B * _N) // _NW
_CHUNKS_PER_W = _TOK_PER_W // _CHUNK
_WPB = _NW // _B
_GROUPS = _CHUNK // _LANES


def _build_sc_embed():
    mesh = plsc.VectorSubcoreMesh(core_axis_name="c", subcore_axis_name="s")

    @functools.partial(
        pl.kernel,
        mesh=mesh,
        compiler_params=pltpu.CompilerParams(
            use_tc_tiling_on_sc=False, needs_layout_passes=False
        ),
        out_type=jax.ShapeDtypeStruct((_B, _D, _N), jnp.float32),
        scratch_types=[
            pltpu.VMEM((_CHUNK,), jnp.int32),
            pltpu.VMEM((_CHUNK, _D), jnp.float32),
            pltpu.VMEM((_D, _CHUNK), jnp.float32),
            pltpu.SemaphoreType.DMA,
        ],
    )
    def k(seq_hbm, emb_hbm, out_hbm, idx_v, g_v, t_v, sem):
        wid = lax.axis_index("s") * _NC + lax.axis_index("c")
        b = wid // _WPB
        h = wid % _WPB
        lane = lax.iota(jnp.int32, _LANES)
        rows = [lane + _LANES * j for j in range(_GROUPS)]
        for c in range(_CHUNKS_PER_W):
            chunk = h * _CHUNKS_PER_W + c
            pltpu.sync_copy(seq_hbm.at[b, chunk], idx_v)
            pltpu.async_copy(emb_hbm.at[idx_v], g_v, sem).wait()

            def body(d, carry):
                dcol = jnp.full((_LANES,), 0, jnp.int32) + d
                for j in range(_GROUPS):
                    vals = plsc.load_gather(g_v, [rows[j], dcol])
                    t_v[d, pl.ds(j * _LANES, _LANES)] = vals
                return carry

            lax.fori_loop(0, _D, body, 0)
            pltpu.sync_copy(
                t_v, out_hbm.at[b, :, pl.ds(chunk * _CHUNK, _CHUNK)]
            )

    return k


_sc_embed = _build_sc_embed()


def kernel(seq, embedding):
    seq3 = seq.astype(jnp.int32).reshape(_B, _N // _CHUNK, _CHUNK)
    out = _sc_embed(seq3, embedding)
    return out.reshape(_B, _D, _HW, _HW)


# speedup vs baseline: 1.1659x; 1.1659x over previous
import functools

import jax
import jax.numpy as jnp
from jax import lax
from jax.experimental import pallas as pl
from jax.experimental.pallas import tpu as pltpu
from jax.experimental.pallas import tpu_sc as plsc

_D = 256
_B = 16
_N = 1024
_HW = 32
_CHUNK = 128
_LANES = 16
_NC = 2
_NS = 16
_NW = _NC * _NS
_TOK_PER_W = (_B * _N) // _NW
_CHUNKS_PER_W = _TOK_PER_W // _CHUNK
_WPB = _NW // _B
_GROUPS = _CHUNK // _LANES


def _build_sc_embed():
    mesh = plsc.VectorSubcoreMesh(core_axis_name="c", subcore_axis_name="s")

    @functools.partial(
        pl.kernel,
        mesh=mesh,
        compiler_params=pltpu.CompilerParams(needs_layout_passes=False),
        out_type=jax.ShapeDtypeStruct((_B, _D, _N), jnp.float32),
        scratch_types=[
            pltpu.VMEM((_CHUNK,), jnp.int32),
            pltpu.VMEM((_CHUNK, _D), jnp.float32),
            pltpu.VMEM((_D, _CHUNK), jnp.float32),
            pltpu.SemaphoreType.DMA,
        ],
    )
    def k(seq_hbm, emb_hbm, out_hbm, idx_v, g_v, t_v, sem):
        wid = lax.axis_index("s") * _NC + lax.axis_index("c")
        b = wid // _WPB
        h = wid % _WPB
        lane = lax.iota(jnp.int32, _LANES)
        rows = [lane + _LANES * j for j in range(_GROUPS)]
        for c in range(_CHUNKS_PER_W):
            chunk = h * _CHUNKS_PER_W + c
            pltpu.sync_copy(seq_hbm.at[b, chunk], idx_v)
            pltpu.async_copy(emb_hbm.at[idx_v], g_v, sem).wait()

            def body(d, carry):
                dcol = jnp.full((_LANES,), 0, jnp.int32) + d
                for j in range(_GROUPS):
                    vals = plsc.load_gather(g_v, [rows[j], dcol])
                    t_v[d, pl.ds(j * _LANES, _LANES)] = vals
                return carry

            lax.fori_loop(0, _D, body, 0)
            pltpu.sync_copy(
                t_v, out_hbm.at[b, :, pl.ds(chunk * _CHUNK, _CHUNK)]
            )

    return k


_sc_embed = _build_sc_embed()


def kernel(seq, embedding):
    seq3 = seq.astype(jnp.int32).reshape(_B, _N // _CHUNK, _CHUNK)
    out = _sc_embed(seq3, embedding)
    return out.reshape(_B, _D, _HW, _HW)


# speedup vs baseline: 1.8444x; 1.5819x over previous
import functools

import jax
import jax.numpy as jnp
from jax import lax
from jax.experimental import pallas as pl
from jax.experimental.pallas import tpu as pltpu
from jax.experimental.pallas import tpu_sc as plsc

_D = 256
_B = 16
_N = 1024
_HW = 32
_CHUNK = 128
_LANES = 16
_NC = 2
_NS = 16
_NW = _NC * _NS
_TOK_PER_W = (_B * _N) // _NW
_CHUNKS_PER_W = _TOK_PER_W // _CHUNK
_WPB = _NW // _B
_GROUPS = _CHUNK // _LANES
_DHALF = _D // 2


def _build_sc_embed():
    mesh = plsc.VectorSubcoreMesh(core_axis_name="c", subcore_axis_name="s")

    @functools.partial(
        pl.kernel,
        mesh=mesh,
        compiler_params=pltpu.CompilerParams(needs_layout_passes=False),
        out_type=jax.ShapeDtypeStruct((_B, _D, _N), jnp.float32),
        scratch_types=[
            pltpu.VMEM((_CHUNKS_PER_W, _CHUNK), jnp.int32),
            pltpu.VMEM((2, _CHUNK, _D), jnp.float32),
            pltpu.VMEM((2, _DHALF, _CHUNK), jnp.float32),
            pltpu.SemaphoreType.DMA,
            pltpu.SemaphoreType.DMA,
            pltpu.SemaphoreType.DMA,
            pltpu.SemaphoreType.DMA,
        ],
    )
    def k(seq_hbm, emb_hbm, out_hbm, idx_v, g_v, t_v, sg0, sg1, sw0, sw1):
        wid = lax.axis_index("s") * _NC + lax.axis_index("c")
        b = wid // _WPB
        h = wid % _WPB
        lane = lax.iota(jnp.int32, _LANES)
        rows = [lane + _LANES * j for j in range(_GROUPS)]
        sg = [sg0, sg1]
        sw = [sw0, sw1]

        pltpu.sync_copy(
            seq_hbm.at[b, pl.ds(h * _CHUNKS_PER_W, _CHUNKS_PER_W)], idx_v
        )

        def start_gather(c):
            return pltpu.async_copy(
                emb_hbm.at[idx_v.at[c]], g_v.at[c % 2], sg[c % 2]
            )

        def transpose_half(c, dh, tbuf):
            g = g_v.at[c % 2]
            t = t_v.at[tbuf]
            d0 = dh * _DHALF

            @plsc.parallel_loop(0, _DHALF, unroll=2)
            def body(d):
                dcol = jnp.full((_LANES,), d0, jnp.int32) + d
                for j in range(_GROUPS):
                    vals = plsc.load_gather(g, [rows[j], dcol])
                    t[d, pl.ds(j * _LANES, _LANES)] = vals

        def start_write(c, dh, tbuf):
            chunk = h * _CHUNKS_PER_W + c
            return pltpu.async_copy(
                t_v.at[tbuf],
                out_hbm.at[
                    b,
                    pl.ds(dh * _DHALF, _DHALF),
                    pl.ds(chunk * _CHUNK, _CHUNK),
                ],
                sw[tbuf],
            )

        steps = [(c, dh) for c in range(_CHUNKS_PER_W) for dh in range(2)]
        writes = [None] * len(steps)
        gathers = [None] * _CHUNKS_PER_W
        gathers[0] = start_gather(0)
        for i, (c, dh) in enumerate(steps):
            if dh == 0:
                if c + 1 < _CHUNKS_PER_W:
                    gathers[c + 1] = start_gather(c + 1)
                gathers[c].wait()
            tbuf = i % 2
            if i >= 2:
                writes[i - 2].wait()
            transpose_half(c, dh, tbuf)
            writes[i] = start_write(c, dh, tbuf)
        writes[-2].wait()
        writes[-1].wait()

    return k


_sc_embed = _build_sc_embed()


def kernel(seq, embedding):
    seq3 = seq.astype(jnp.int32).reshape(_B, _N // _CHUNK, _CHUNK)
    out = _sc_embed(seq3, embedding)
    return out.reshape(_B, _D, _HW, _HW)


# speedup vs baseline: 5.2928x; 2.8696x over previous
import functools

import jax
import jax.numpy as jnp
from jax import lax
from jax.experimental import pallas as pl
from jax.experimental.pallas import tpu as pltpu
from jax.experimental.pallas import tpu_sc as plsc

_D = 256
_B = 16
_N = 1024
_HW = 32
_CHUNK = 128
_NC = 2
_NS = 16
_NW = _NC * _NS
_TOK = _B * _N
_CHUNKS = _TOK // _CHUNK
_CHUNKS_PER_W = _CHUNKS // _NW
_NBUF = 3


def _build_sc_gather():
    mesh = plsc.VectorSubcoreMesh(core_axis_name="c", subcore_axis_name="s")

    @functools.partial(
        pl.kernel,
        mesh=mesh,
        compiler_params=pltpu.CompilerParams(
            needs_layout_passes=False,
            disable_bounds_checks=True,
            disable_semaphore_checks=True,
        ),
        out_type=jax.ShapeDtypeStruct((_TOK, _D), jnp.float32),
        scratch_types=[
            pltpu.VMEM((_CHUNKS_PER_W, _CHUNK), jnp.int32),
            pltpu.VMEM((_NBUF, _CHUNK, _D), jnp.float32),
            pltpu.SemaphoreType.DMA,
            pltpu.SemaphoreType.DMA,
            pltpu.SemaphoreType.DMA,
            pltpu.SemaphoreType.DMA,
            pltpu.SemaphoreType.DMA,
            pltpu.SemaphoreType.DMA,
        ],
    )
    def k(seq_hbm, emb_hbm, out_hbm, idx_v, g_v, sg0, sg1, sg2, sw0, sw1, sw2):
        wid = lax.axis_index("s") * _NC + lax.axis_index("c")
        base = wid * _CHUNKS_PER_W
        sg = [sg0, sg1, sg2]
        sw = [sw0, sw1, sw2]

        pltpu.sync_copy(seq_hbm.at[pl.ds(base, _CHUNKS_PER_W)], idx_v)

        def start_gather(c):
            return pltpu.async_copy(
                emb_hbm.at[idx_v.at[c]], g_v.at[c % _NBUF], sg[c % _NBUF]
            )

        def start_write(c):
            return pltpu.async_copy(
                g_v.at[c % _NBUF],
                out_hbm.at[pl.ds((base + c) * _CHUNK, _CHUNK)],
                sw[c % _NBUF],
            )

        gathers = [None] * _CHUNKS_PER_W
        writes = [None] * _CHUNKS_PER_W
        for c in range(min(_NBUF - 1, _CHUNKS_PER_W)):
            gathers[c] = start_gather(c)
        for c in range(_CHUNKS_PER_W):
            gathers[c].wait()
            writes[c] = start_write(c)
            nxt = c + _NBUF - 1
            if nxt < _CHUNKS_PER_W:
                prev = nxt - _NBUF
                if prev >= 0:
                    writes[prev].wait()
                gathers[nxt] = start_gather(nxt)
        for c in range(max(0, _CHUNKS_PER_W - _NBUF), _CHUNKS_PER_W):
            if writes[c] is not None:
                writes[c].wait()

    return k


_sc_gather = _build_sc_gather()


def kernel(seq, embedding):
    seq2 = seq.astype(jnp.int32).reshape(_CHUNKS, _CHUNK)
    rows = _sc_gather(seq2, embedding)
    out = rows.reshape(_B, _HW, _HW, _D)
    return jnp.transpose(out, (0, 3, 1, 2))
